# baseline (speedup 1.0000x reference)
import functools

import jax
import jax.numpy as jnp
from jax import lax
from jax.experimental import pallas as pl
from jax.experimental.pallas import tpu as pltpu
from jax.experimental.pallas import tpu_sc as plsc

D_MODEL = 256
SEQ = 64
BATCH = 4096
LANES = 16
NCHUNK = D_MODEL // LANES


NW = 32
BPW = BATCH // NW
PAIR = 2
NPAIR = BPW // PAIR
NBUF = 3


def _sc_body(x_hbm, abs_hbm, file_hbm, rank_hbm, diag_hbm, anti_hbm, out_hbm,
             idx_v, pos_v, xb0, xb1, xb2,
             in0, in1, in2, out0, out1, out2, gsem):
    cid = lax.axis_index("c")
    sid = lax.axis_index("s")
    wid = sid * 2 + cid
    base = wid * BPW
    xbs = [xb0, xb1, xb2]
    insems = [in0, in1, in2]
    outsems = [out0, out1, out2]

    pltpu.sync_copy(abs_hbm.at[0], pos_v)

    def add_tmp_into_pos():
        def srow(s, carry):
            for ch in range(NCHUNK):
                sl = pl.ds(ch * LANES, LANES)
                pos_v[s, sl] = pos_v[s, sl] + xb0[0, s, sl]
            return carry
        lax.fori_loop(0, SEQ, srow, 0)

    _k3 = jnp.full((LANES,), 3, dtype=jnp.int32)
    _k7 = jnp.full((LANES,), 7, dtype=jnp.int32)
    for table, fn in (
        (file_hbm, lambda p: p & _k7),
        (rank_hbm, lambda p: p >> _k3),
        (diag_hbm, lambda p: (p >> _k3) + (p & _k7)),
        (anti_hbm, lambda p: (p >> _k3) - (p & _k7) + _k7),
    ):
        for ch in range(SEQ // LANES):
            c16 = jnp.full((LANES,), ch * LANES, dtype=jnp.int32)
            p = lax.iota(jnp.int32, LANES) + c16
            idx_v[pl.ds(ch * LANES, LANES)] = fn(p)
        pltpu.async_copy(table.at[idx_v], xb0.at[0], gsem).wait()
        add_tmp_into_pos()

    def turn(e, b):
        buf = xbs[b]
        pltpu.make_async_copy(
            x_hbm.at[pl.ds(base + e * PAIR, PAIR)], buf, insems[b]).wait()

        def srow(s, c2):
            prow = [pos_v[s, pl.ds(ch * LANES, LANES)] for ch in range(NCHUNK)]
            for el in range(PAIR):
                for ch in range(NCHUNK):
                    sl = pl.ds(ch * LANES, LANES)
                    buf[el, s, sl] = buf[el, s, sl] + prow[ch]
            return c2
        lax.fori_loop(0, SEQ, srow, 0)
        pltpu.async_copy(
            buf, out_hbm.at[pl.ds(base + e * PAIR, PAIR)], outsems[b])

        bp = (b - 1) % NBUF
        ep = e - 1
        @pl.when((ep >= 0) & (ep + NBUF < NPAIR))
        def _():
            pltpu.make_async_copy(
                xbs[bp], out_hbm.at[pl.ds(base + ep * PAIR, PAIR)],
                outsems[bp]).wait()
            pltpu.async_copy(
                x_hbm.at[pl.ds(base + (ep + NBUF) * PAIR, PAIR)],
                xbs[bp], insems[bp])

    for b in range(NBUF):
        pltpu.async_copy(
            x_hbm.at[pl.ds(base + b * PAIR, PAIR)], xbs[b], insems[b])

    NFULL = NPAIR // NBUF
    NTAIL = NPAIR - NFULL * NBUF

    def ring_step(i, carry):
        for b in range(NBUF):
            turn(i * NBUF + b, b)
        return carry
    lax.fori_loop(0, NFULL, ring_step, 0)
    for t in range(NTAIL):
        turn(NFULL * NBUF + t, t)

    for k in range(NBUF):
        e = NPAIR - NBUF + k
        b = e % NBUF
        pltpu.make_async_copy(
            xbs[b], out_hbm.at[pl.ds(base + e * PAIR, PAIR)],
            outsems[b]).wait()


_sc_kernel = functools.partial(
    pl.kernel,
    out_type=jax.ShapeDtypeStruct((BATCH, SEQ, D_MODEL), jnp.float32),
    mesh=plsc.VectorSubcoreMesh(core_axis_name="c", subcore_axis_name="s"),
    scratch_types=[
        pltpu.VMEM((SEQ,), jnp.int32),
        pltpu.VMEM((SEQ, D_MODEL), jnp.float32),
    ] + [pltpu.VMEM((PAIR, SEQ, D_MODEL), jnp.float32)] * 3
      + [pltpu.SemaphoreType.DMA] * 7,
)(_sc_body)


BATCH_BLOCK = 128


def _tc_body(x_ref, abs_ref, file_ref, rank_ref, diag_ref, anti_ref, o_ref):
    file_emb = jnp.tile(file_ref[...], (8, 1))
    rank_emb = jnp.repeat(rank_ref[...], 8, axis=0)
    row = jax.lax.broadcasted_iota(jnp.int32, (SEQ, 15), 0)
    col = jax.lax.broadcasted_iota(jnp.int32, (SEQ, 15), 1)
    diag_oh = (col == row // 8 + row % 8).astype(jnp.float32)
    anti_oh = (col == row // 8 - row % 8 + 7).astype(jnp.float32)
    diag_emb = jnp.dot(diag_oh, diag_ref[...], preferred_element_type=jnp.float32,
                       precision=jax.lax.Precision.HIGHEST)
    anti_emb = jnp.dot(anti_oh, anti_ref[...], preferred_element_type=jnp.float32,
                       precision=jax.lax.Precision.HIGHEST)
    pos = abs_ref[0] + file_emb + rank_emb + diag_emb + anti_emb
    o_ref[...] = x_ref[...] + pos[None, :, :]


def _tc_kernel(x, absolute_pos_embedding, file_table, rank_table, diag_table, anti_diag_table):
    batch, seq, d = x.shape
    return pl.pallas_call(
        _tc_body,
        grid=(batch // BATCH_BLOCK,),
        in_specs=[
            pl.BlockSpec((BATCH_BLOCK, seq, d), lambda i: (i, 0, 0)),
            pl.BlockSpec((1, seq, d), lambda i: (0, 0, 0)),
            pl.BlockSpec((8, d), lambda i: (0, 0)),
            pl.BlockSpec((8, d), lambda i: (0, 0)),
            pl.BlockSpec((15, d), lambda i: (0, 0)),
            pl.BlockSpec((15, d), lambda i: (0, 0)),
        ],
        out_specs=pl.BlockSpec((BATCH_BLOCK, seq, d), lambda i: (i, 0, 0)),
        out_shape=jax.ShapeDtypeStruct(x.shape, x.dtype),
    )(x, absolute_pos_embedding, file_table, rank_table, diag_table, anti_diag_table)


@jax.jit
def kernel(x, absolute_pos_embedding, file_table, rank_table, diag_table, anti_diag_table):
    return _sc_kernel(x, absolute_pos_embedding, file_table, rank_table,
                      diag_table, anti_diag_table)


# speedup vs baseline: 1.0140x; 1.0140x over previous
import functools

import jax
import jax.numpy as jnp
from jax import lax
from jax.experimental import pallas as pl
from jax.experimental.pallas import tpu as pltpu
from jax.experimental.pallas import tpu_sc as plsc

D_MODEL = 256
SEQ = 64
BATCH = 4096
LANES = 16
NCHUNK = D_MODEL // LANES


NW = 32
BPW = BATCH // NW
PAIR = 2
NPAIR = BPW // PAIR
NBUF = 3


def _sc_body(x_hbm, abs_hbm, file_hbm, rank_hbm, diag_hbm, anti_hbm, out_hbm,
             idx_v, pos_v, xb0, xb1, xb2,
             in0, in1, in2, out0, out1, out2, gsem):
    cid = lax.axis_index("c")
    sid = lax.axis_index("s")
    wid = sid * 2 + cid
    base = wid * BPW
    xbs = [xb0, xb1, xb2]
    insems = [in0, in1, in2]
    outsems = [out0, out1, out2]

    pltpu.sync_copy(abs_hbm.at[0], pos_v)

    def add_tmp_into_pos():
        def srow(s, carry):
            for ch in range(NCHUNK):
                sl = pl.ds(ch * LANES, LANES)
                pos_v[s, sl] = pos_v[s, sl] + xb0[0, s, sl]
            return carry
        lax.fori_loop(0, SEQ, srow, 0)

    _k3 = jnp.full((LANES,), 3, dtype=jnp.int32)
    _k7 = jnp.full((LANES,), 7, dtype=jnp.int32)
    for table, fn in (
        (file_hbm, lambda p: p & _k7),
        (rank_hbm, lambda p: p >> _k3),
        (diag_hbm, lambda p: (p >> _k3) + (p & _k7)),
        (anti_hbm, lambda p: (p >> _k3) - (p & _k7) + _k7),
    ):
        for ch in range(SEQ // LANES):
            c16 = jnp.full((LANES,), ch * LANES, dtype=jnp.int32)
            p = lax.iota(jnp.int32, LANES) + c16
            idx_v[pl.ds(ch * LANES, LANES)] = fn(p)
        pltpu.async_copy(table.at[idx_v], xb0.at[0], gsem).wait()
        add_tmp_into_pos()

    def turn(e, b):
        buf = xbs[b]
        pltpu.make_async_copy(
            x_hbm.at[pl.ds(base + e * PAIR, PAIR)], buf, insems[b]).wait()

        def srow(s, c2):
            prow = [pos_v[s, pl.ds(ch * LANES, LANES)] for ch in range(NCHUNK)]
            for el in range(PAIR):
                for ch in range(NCHUNK):
                    sl = pl.ds(ch * LANES, LANES)
                    buf[el, s, sl] = buf[el, s, sl] + prow[ch]
            return c2
        if True:
            pass
        else:
            lax.fori_loop(0, SEQ, srow, 0)
        pltpu.async_copy(
            buf, out_hbm.at[pl.ds(base + e * PAIR, PAIR)], outsems[b])

        bp = (b - 1) % NBUF
        ep = e - 1
        @pl.when((ep >= 0) & (ep + NBUF < NPAIR))
        def _():
            pltpu.make_async_copy(
                xbs[bp], out_hbm.at[pl.ds(base + ep * PAIR, PAIR)],
                outsems[bp]).wait()
            pltpu.async_copy(
                x_hbm.at[pl.ds(base + (ep + NBUF) * PAIR, PAIR)],
                xbs[bp], insems[bp])

    for b in range(NBUF):
        pltpu.async_copy(
            x_hbm.at[pl.ds(base + b * PAIR, PAIR)], xbs[b], insems[b])

    NFULL = NPAIR // NBUF
    NTAIL = NPAIR - NFULL * NBUF

    def ring_step(i, carry):
        for b in range(NBUF):
            turn(i * NBUF + b, b)
        return carry
    lax.fori_loop(0, NFULL, ring_step, 0)
    for t in range(NTAIL):
        turn(NFULL * NBUF + t, t)

    for k in range(NBUF):
        e = NPAIR - NBUF + k
        b = e % NBUF
        pltpu.make_async_copy(
            xbs[b], out_hbm.at[pl.ds(base + e * PAIR, PAIR)],
            outsems[b]).wait()


_sc_kernel = functools.partial(
    pl.kernel,
    out_type=jax.ShapeDtypeStruct((BATCH, SEQ, D_MODEL), jnp.float32),
    mesh=plsc.VectorSubcoreMesh(core_axis_name="c", subcore_axis_name="s"),
    scratch_types=[
        pltpu.VMEM((SEQ,), jnp.int32),
        pltpu.VMEM((SEQ, D_MODEL), jnp.float32),
    ] + [pltpu.VMEM((PAIR, SEQ, D_MODEL), jnp.float32)] * 3
      + [pltpu.SemaphoreType.DMA] * 7,
)(_sc_body)


BATCH_BLOCK = 128


def _tc_body(x_ref, abs_ref, file_ref, rank_ref, diag_ref, anti_ref, o_ref):
    file_emb = jnp.tile(file_ref[...], (8, 1))
    rank_emb = jnp.repeat(rank_ref[...], 8, axis=0)
    row = jax.lax.broadcasted_iota(jnp.int32, (SEQ, 15), 0)
    col = jax.lax.broadcasted_iota(jnp.int32, (SEQ, 15), 1)
    diag_oh = (col == row // 8 + row % 8).astype(jnp.float32)
    anti_oh = (col == row // 8 - row % 8 + 7).astype(jnp.float32)
    diag_emb = jnp.dot(diag_oh, diag_ref[...], preferred_element_type=jnp.float32,
                       precision=jax.lax.Precision.HIGHEST)
    anti_emb = jnp.dot(anti_oh, anti_ref[...], preferred_element_type=jnp.float32,
                       precision=jax.lax.Precision.HIGHEST)
    pos = abs_ref[0] + file_emb + rank_emb + diag_emb + anti_emb
    o_ref[...] = x_ref[...] + pos[None, :, :]


def _tc_kernel(x, absolute_pos_embedding, file_table, rank_table, diag_table, anti_diag_table):
    batch, seq, d = x.shape
    return pl.pallas_call(
        _tc_body,
        grid=(batch // BATCH_BLOCK,),
        in_specs=[
            pl.BlockSpec((BATCH_BLOCK, seq, d), lambda i: (i, 0, 0)),
            pl.BlockSpec((1, seq, d), lambda i: (0, 0, 0)),
            pl.BlockSpec((8, d), lambda i: (0, 0)),
            pl.BlockSpec((8, d), lambda i: (0, 0)),
            pl.BlockSpec((15, d), lambda i: (0, 0)),
            pl.BlockSpec((15, d), lambda i: (0, 0)),
        ],
        out_specs=pl.BlockSpec((BATCH_BLOCK, seq, d), lambda i: (i, 0, 0)),
        out_shape=jax.ShapeDtypeStruct(x.shape, x.dtype),
    )(x, absolute_pos_embedding, file_table, rank_table, diag_table, anti_diag_table)


@jax.jit
def kernel(x, absolute_pos_embedding, file_table, rank_table, diag_table, anti_diag_table):
    return _sc_kernel(x, absolute_pos_embedding, file_table, rank_table,
                      diag_table, anti_diag_table)


# speedup vs baseline: 1.2196x; 1.2029x over previous
import functools

import jax
import jax.numpy as jnp
from jax import lax
from jax.experimental import pallas as pl
from jax.experimental.pallas import tpu as pltpu
from jax.experimental.pallas import tpu_sc as plsc

D_MODEL = 256
SEQ = 64
BATCH = 4096
LANES = 16
NCHUNK = D_MODEL // LANES


NW = 32
BPW = BATCH // NW
PAIR = 2
NPAIR = BPW // PAIR
NBUF = 3


def _sc_body(x_hbm, abs_hbm, file_hbm, rank_hbm, diag_hbm, anti_hbm, out_hbm,
             idx_v, pos_v, xb0, xb1, xb2,
             in0, in1, in2, out0, out1, out2, gsem):
    cid = lax.axis_index("c")
    sid = lax.axis_index("s")
    wid = sid * 2 + cid
    base = wid * BPW
    xbs = [xb0, xb1, xb2]
    insems = [in0, in1, in2]
    outsems = [out0, out1, out2]

    pltpu.sync_copy(abs_hbm.at[0], pos_v)

    def add_tmp_into_pos():
        def srow(s, carry):
            for ch in range(NCHUNK):
                sl = pl.ds(ch * LANES, LANES)
                pos_v[s, sl] = pos_v[s, sl] + xb0[0, s, sl]
            return carry
        lax.fori_loop(0, SEQ, srow, 0)

    _k3 = jnp.full((LANES,), 3, dtype=jnp.int32)
    _k7 = jnp.full((LANES,), 7, dtype=jnp.int32)
    for table, fn in (
        (file_hbm, lambda p: p & _k7),
        (rank_hbm, lambda p: p >> _k3),
        (diag_hbm, lambda p: (p >> _k3) + (p & _k7)),
        (anti_hbm, lambda p: (p >> _k3) - (p & _k7) + _k7),
    ):
        for ch in range(SEQ // LANES):
            c16 = jnp.full((LANES,), ch * LANES, dtype=jnp.int32)
            p = lax.iota(jnp.int32, LANES) + c16
            idx_v[pl.ds(ch * LANES, LANES)] = fn(p)
        pltpu.async_copy(table.at[idx_v], xb0.at[0], gsem).wait()
        add_tmp_into_pos()

    def turn(e, b):
        buf = xbs[b]
        pltpu.make_async_copy(
            x_hbm.at[pl.ds(base + e * PAIR, PAIR)], buf, insems[b]).wait()

        def srow(s, c2):
            prow = [pos_v[s, pl.ds(ch * LANES, LANES)] for ch in range(NCHUNK)]
            for el in range(PAIR):
                for ch in range(NCHUNK):
                    sl = pl.ds(ch * LANES, LANES)
                    buf[el, s, sl] = buf[el, s, sl] + prow[ch]
            return c2
        lax.fori_loop(0, SEQ, srow, 0)
        pltpu.async_copy(
            buf, out_hbm.at[pl.ds(base + e * PAIR, PAIR)], outsems[b])

        bp = (b - 1) % NBUF
        ep = e - 1
        @pl.when((ep >= 0) & (ep + NBUF < NPAIR))
        def _():
            pltpu.make_async_copy(
                xbs[bp], out_hbm.at[pl.ds(base + ep * PAIR, PAIR)],
                outsems[bp]).wait()
            pltpu.async_copy(
                x_hbm.at[pl.ds(base + (ep + NBUF) * PAIR, PAIR)],
                xbs[bp], insems[bp])

    for b in range(NBUF):
        pltpu.async_copy(
            x_hbm.at[pl.ds(base + b * PAIR, PAIR)], xbs[b], insems[b])

    NFULL = NPAIR // NBUF
    NTAIL = NPAIR - NFULL * NBUF

    def ring_step(i, carry):
        for b in range(NBUF):
            turn(i * NBUF + b, b)
        return carry
    lax.fori_loop(0, NFULL, ring_step, 0)
    for t in range(NTAIL):
        turn(NFULL * NBUF + t, t)

    for k in range(NBUF):
        e = NPAIR - NBUF + k
        b = e % NBUF
        pltpu.make_async_copy(
            xbs[b], out_hbm.at[pl.ds(base + e * PAIR, PAIR)],
            outsems[b]).wait()


_sc_kernel = functools.partial(
    pl.kernel,
    out_type=jax.ShapeDtypeStruct((BATCH, SEQ, D_MODEL), jnp.float32),
    mesh=plsc.VectorSubcoreMesh(core_axis_name="c", subcore_axis_name="s"),
    scratch_types=[
        pltpu.VMEM((SEQ,), jnp.int32),
        pltpu.VMEM((SEQ, D_MODEL), jnp.float32),
    ] + [pltpu.VMEM((PAIR, SEQ, D_MODEL), jnp.float32)] * 3
      + [pltpu.SemaphoreType.DMA] * 7,
)(_sc_body)


def _sc_pos_body(abs_hbm, file_hbm, rank_hbm, diag_hbm, anti_hbm, pos_hbm,
                 idx_v, pos_v, tmp_v, gsem):
    cid = lax.axis_index("c")
    sid = lax.axis_index("s")
    wid = sid * 2 + cid

    @pl.when(wid == 0)
    def _():
        pltpu.sync_copy(abs_hbm.at[0], pos_v)

        def add_tmp_into_pos():
            def srow(s, carry):
                for ch in range(NCHUNK):
                    sl = pl.ds(ch * LANES, LANES)
                    pos_v[s, sl] = pos_v[s, sl] + tmp_v[s, sl]
                return carry
            lax.fori_loop(0, SEQ, srow, 0)

        _k3 = jnp.full((LANES,), 3, dtype=jnp.int32)
        _k7 = jnp.full((LANES,), 7, dtype=jnp.int32)
        for table, fn in (
            (file_hbm, lambda p: p & _k7),
            (rank_hbm, lambda p: p >> _k3),
            (diag_hbm, lambda p: (p >> _k3) + (p & _k7)),
            (anti_hbm, lambda p: (p >> _k3) - (p & _k7) + _k7),
        ):
            for ch in range(SEQ // LANES):
                c16 = jnp.full((LANES,), ch * LANES, dtype=jnp.int32)
                p = lax.iota(jnp.int32, LANES) + c16
                idx_v[pl.ds(ch * LANES, LANES)] = fn(p)
            pltpu.async_copy(table.at[idx_v], tmp_v, gsem).wait()
            add_tmp_into_pos()
        pltpu.sync_copy(pos_v, pos_hbm)


_sc_pos_kernel = functools.partial(
    pl.kernel,
    out_type=jax.ShapeDtypeStruct((SEQ, D_MODEL), jnp.float32),
    mesh=plsc.VectorSubcoreMesh(core_axis_name="c", subcore_axis_name="s"),
    scratch_types=[
        pltpu.VMEM((SEQ,), jnp.int32),
        pltpu.VMEM((SEQ, D_MODEL), jnp.float32),
        pltpu.VMEM((SEQ, D_MODEL), jnp.float32),
        pltpu.SemaphoreType.DMA,
    ],
)(_sc_pos_body)


def _tc_add_body(x_ref, pos_ref, o_ref):
    o_ref[...] = x_ref[...] + pos_ref[...][None, :, :]


def _hybrid_kernel(x, absolute_pos_embedding, file_table, rank_table,
                   diag_table, anti_diag_table):
    pos = _sc_pos_kernel(absolute_pos_embedding, file_table, rank_table,
                         diag_table, anti_diag_table)
    batch, seq, d = x.shape
    return pl.pallas_call(
        _tc_add_body,
        grid=(batch // BATCH_BLOCK,),
        in_specs=[
            pl.BlockSpec((BATCH_BLOCK, seq, d), lambda i: (i, 0, 0)),
            pl.BlockSpec((seq, d), lambda i: (0, 0)),
        ],
        out_specs=pl.BlockSpec((BATCH_BLOCK, seq, d), lambda i: (i, 0, 0)),
        out_shape=jax.ShapeDtypeStruct(x.shape, x.dtype),
    )(x, pos)


BATCH_BLOCK = 128


def _tc_body(x_ref, abs_ref, file_ref, rank_ref, diag_ref, anti_ref, o_ref):
    file_emb = jnp.tile(file_ref[...], (8, 1))
    rank_emb = jnp.repeat(rank_ref[...], 8, axis=0)
    row = jax.lax.broadcasted_iota(jnp.int32, (SEQ, 15), 0)
    col = jax.lax.broadcasted_iota(jnp.int32, (SEQ, 15), 1)
    diag_oh = (col == row // 8 + row % 8).astype(jnp.float32)
    anti_oh = (col == row // 8 - row % 8 + 7).astype(jnp.float32)
    diag_emb = jnp.dot(diag_oh, diag_ref[...], preferred_element_type=jnp.float32,
                       precision=jax.lax.Precision.HIGHEST)
    anti_emb = jnp.dot(anti_oh, anti_ref[...], preferred_element_type=jnp.float32,
                       precision=jax.lax.Precision.HIGHEST)
    pos = abs_ref[0] + file_emb + rank_emb + diag_emb + anti_emb
    o_ref[...] = x_ref[...] + pos[None, :, :]


def _tc_kernel(x, absolute_pos_embedding, file_table, rank_table, diag_table, anti_diag_table):
    batch, seq, d = x.shape
    return pl.pallas_call(
        _tc_body,
        grid=(batch // BATCH_BLOCK,),
        in_specs=[
            pl.BlockSpec((BATCH_BLOCK, seq, d), lambda i: (i, 0, 0)),
            pl.BlockSpec((1, seq, d), lambda i: (0, 0, 0)),
            pl.BlockSpec((8, d), lambda i: (0, 0)),
            pl.BlockSpec((8, d), lambda i: (0, 0)),
            pl.BlockSpec((15, d), lambda i: (0, 0)),
            pl.BlockSpec((15, d), lambda i: (0, 0)),
        ],
        out_specs=pl.BlockSpec((BATCH_BLOCK, seq, d), lambda i: (i, 0, 0)),
        out_shape=jax.ShapeDtypeStruct(x.shape, x.dtype),
    )(x, absolute_pos_embedding, file_table, rank_table, diag_table, anti_diag_table)


@jax.jit
def kernel(x, absolute_pos_embedding, file_table, rank_table, diag_table, anti_diag_table):
    return _hybrid_kernel(x, absolute_pos_embedding, file_table, rank_table,
                          diag_table, anti_diag_table)


# speedup vs baseline: 1.4365x; 1.1778x over previous
import functools

import jax
import jax.numpy as jnp
from jax import lax
from jax.experimental import pallas as pl
from jax.experimental.pallas import tpu as pltpu
from jax.experimental.pallas import tpu_sc as plsc

D_MODEL = 256
SEQ = 64
BATCH = 4096
LANES = 16
NCHUNK = D_MODEL // LANES


NW = 32
BPW = BATCH // NW
PAIR = 2
NPAIR = BPW // PAIR
NBUF = 3


def _sc_body(x_hbm, abs_hbm, file_hbm, rank_hbm, diag_hbm, anti_hbm, out_hbm,
             idx_v, pos_v, xb0, xb1, xb2,
             in0, in1, in2, out0, out1, out2, gsem):
    cid = lax.axis_index("c")
    sid = lax.axis_index("s")
    wid = sid * 2 + cid
    base = wid * BPW
    xbs = [xb0, xb1, xb2]
    insems = [in0, in1, in2]
    outsems = [out0, out1, out2]

    pltpu.sync_copy(abs_hbm.at[0], pos_v)

    def add_tmp_into_pos():
        def srow(s, carry):
            for ch in range(NCHUNK):
                sl = pl.ds(ch * LANES, LANES)
                pos_v[s, sl] = pos_v[s, sl] + xb0[0, s, sl]
            return carry
        lax.fori_loop(0, SEQ, srow, 0)

    _k3 = jnp.full((LANES,), 3, dtype=jnp.int32)
    _k7 = jnp.full((LANES,), 7, dtype=jnp.int32)
    for table, fn in (
        (file_hbm, lambda p: p & _k7),
        (rank_hbm, lambda p: p >> _k3),
        (diag_hbm, lambda p: (p >> _k3) + (p & _k7)),
        (anti_hbm, lambda p: (p >> _k3) - (p & _k7) + _k7),
    ):
        for ch in range(SEQ // LANES):
            c16 = jnp.full((LANES,), ch * LANES, dtype=jnp.int32)
            p = lax.iota(jnp.int32, LANES) + c16
            idx_v[pl.ds(ch * LANES, LANES)] = fn(p)
        pltpu.async_copy(table.at[idx_v], xb0.at[0], gsem).wait()
        add_tmp_into_pos()

    def turn(e, b):
        buf = xbs[b]
        pltpu.make_async_copy(
            x_hbm.at[pl.ds(base + e * PAIR, PAIR)], buf, insems[b]).wait()

        def srow(s, c2):
            prow = [pos_v[s, pl.ds(ch * LANES, LANES)] for ch in range(NCHUNK)]
            for el in range(PAIR):
                for ch in range(NCHUNK):
                    sl = pl.ds(ch * LANES, LANES)
                    buf[el, s, sl] = buf[el, s, sl] + prow[ch]
            return c2
        lax.fori_loop(0, SEQ, srow, 0)
        pltpu.async_copy(
            buf, out_hbm.at[pl.ds(base + e * PAIR, PAIR)], outsems[b])

        bp = (b - 1) % NBUF
        ep = e - 1
        @pl.when((ep >= 0) & (ep + NBUF < NPAIR))
        def _():
            pltpu.make_async_copy(
                xbs[bp], out_hbm.at[pl.ds(base + ep * PAIR, PAIR)],
                outsems[bp]).wait()
            pltpu.async_copy(
                x_hbm.at[pl.ds(base + (ep + NBUF) * PAIR, PAIR)],
                xbs[bp], insems[bp])

    for b in range(NBUF):
        pltpu.async_copy(
            x_hbm.at[pl.ds(base + b * PAIR, PAIR)], xbs[b], insems[b])

    NFULL = NPAIR // NBUF
    NTAIL = NPAIR - NFULL * NBUF

    def ring_step(i, carry):
        for b in range(NBUF):
            turn(i * NBUF + b, b)
        return carry
    lax.fori_loop(0, NFULL, ring_step, 0)
    for t in range(NTAIL):
        turn(NFULL * NBUF + t, t)

    for k in range(NBUF):
        e = NPAIR - NBUF + k
        b = e % NBUF
        pltpu.make_async_copy(
            xbs[b], out_hbm.at[pl.ds(base + e * PAIR, PAIR)],
            outsems[b]).wait()


_sc_kernel = functools.partial(
    pl.kernel,
    out_type=jax.ShapeDtypeStruct((BATCH, SEQ, D_MODEL), jnp.float32),
    mesh=plsc.VectorSubcoreMesh(core_axis_name="c", subcore_axis_name="s"),
    scratch_types=[
        pltpu.VMEM((SEQ,), jnp.int32),
        pltpu.VMEM((SEQ, D_MODEL), jnp.float32),
    ] + [pltpu.VMEM((PAIR, SEQ, D_MODEL), jnp.float32)] * 3
      + [pltpu.SemaphoreType.DMA] * 7,
)(_sc_body)


def _sc_pos_body(abs_hbm, file_hbm, rank_hbm, diag_hbm, anti_hbm, pos_hbm,
                 idx_v, pos_v, tmp_v, gsem):
    cid = lax.axis_index("c")
    sid = lax.axis_index("s")
    wid = sid * 2 + cid

    @pl.when(wid == 0)
    def _():
        pltpu.sync_copy(abs_hbm.at[0], pos_v)

        def add_tmp_into_pos():
            def srow(s, carry):
                for ch in range(NCHUNK):
                    sl = pl.ds(ch * LANES, LANES)
                    pos_v[s, sl] = pos_v[s, sl] + tmp_v[s, sl]
                return carry
            lax.fori_loop(0, SEQ, srow, 0)

        _k3 = jnp.full((LANES,), 3, dtype=jnp.int32)
        _k7 = jnp.full((LANES,), 7, dtype=jnp.int32)
        for table, fn in (
            (file_hbm, lambda p: p & _k7),
            (rank_hbm, lambda p: p >> _k3),
            (diag_hbm, lambda p: (p >> _k3) + (p & _k7)),
            (anti_hbm, lambda p: (p >> _k3) - (p & _k7) + _k7),
        ):
            for ch in range(SEQ // LANES):
                c16 = jnp.full((LANES,), ch * LANES, dtype=jnp.int32)
                p = lax.iota(jnp.int32, LANES) + c16
                idx_v[pl.ds(ch * LANES, LANES)] = fn(p)
            pltpu.async_copy(table.at[idx_v], tmp_v, gsem).wait()
            add_tmp_into_pos()
        pltpu.sync_copy(pos_v, pos_hbm)


_sc_pos_kernel = functools.partial(
    pl.kernel,
    out_type=jax.ShapeDtypeStruct((SEQ, D_MODEL), jnp.float32),
    mesh=plsc.VectorSubcoreMesh(core_axis_name="c", subcore_axis_name="s"),
    scratch_types=[
        pltpu.VMEM((SEQ,), jnp.int32),
        pltpu.VMEM((SEQ, D_MODEL), jnp.float32),
        pltpu.VMEM((SEQ, D_MODEL), jnp.float32),
        pltpu.SemaphoreType.DMA,
    ],
)(_sc_pos_body)


def _tc_add_body(x_ref, pos_ref, o_ref):
    o_ref[...] = x_ref[...] + pos_ref[...][None, :, :]


def _hybrid_kernel(x, absolute_pos_embedding, file_table, rank_table,
                   diag_table, anti_diag_table):
    pos = _sc_pos_kernel(absolute_pos_embedding, file_table, rank_table,
                         diag_table, anti_diag_table)
    batch, seq, d = x.shape
    return pl.pallas_call(
        _tc_add_body,
        grid=(batch // BATCH_BLOCK,),
        in_specs=[
            pl.BlockSpec((BATCH_BLOCK, seq, d), lambda i: (i, 0, 0)),
            pl.BlockSpec((seq, d), lambda i: (0, 0)),
        ],
        out_specs=pl.BlockSpec((BATCH_BLOCK, seq, d), lambda i: (i, 0, 0)),
        out_shape=jax.ShapeDtypeStruct(x.shape, x.dtype),
    )(x, pos)


BATCH_BLOCK = 64


def _tc_body(x_ref, abs_ref, file_ref, rank_ref, diag_ref, anti_ref, o_ref):
    file_emb = jnp.tile(file_ref[...], (8, 1))
    rank_emb = jnp.repeat(rank_ref[...], 8, axis=0)
    row = jax.lax.broadcasted_iota(jnp.int32, (SEQ, 15), 0)
    col = jax.lax.broadcasted_iota(jnp.int32, (SEQ, 15), 1)
    diag_oh = (col == row // 8 + row % 8).astype(jnp.float32)
    anti_oh = (col == row // 8 - row % 8 + 7).astype(jnp.float32)
    diag_emb = jnp.dot(diag_oh, diag_ref[...], preferred_element_type=jnp.float32,
                       precision=jax.lax.Precision.HIGHEST)
    anti_emb = jnp.dot(anti_oh, anti_ref[...], preferred_element_type=jnp.float32,
                       precision=jax.lax.Precision.HIGHEST)
    pos = abs_ref[0] + file_emb + rank_emb + diag_emb + anti_emb
    o_ref[...] = x_ref[...] + pos[None, :, :]


def _tc_kernel(x, absolute_pos_embedding, file_table, rank_table, diag_table, anti_diag_table):
    batch, seq, d = x.shape
    return pl.pallas_call(
        _tc_body,
        grid=(batch // BATCH_BLOCK,),
        in_specs=[
            pl.BlockSpec((BATCH_BLOCK, seq, d), lambda i: (i, 0, 0)),
            pl.BlockSpec((1, seq, d), lambda i: (0, 0, 0)),
            pl.BlockSpec((8, d), lambda i: (0, 0)),
            pl.BlockSpec((8, d), lambda i: (0, 0)),
            pl.BlockSpec((15, d), lambda i: (0, 0)),
            pl.BlockSpec((15, d), lambda i: (0, 0)),
        ],
        out_specs=pl.BlockSpec((BATCH_BLOCK, seq, d), lambda i: (i, 0, 0)),
        out_shape=jax.ShapeDtypeStruct(x.shape, x.dtype),
    )(x, absolute_pos_embedding, file_table, rank_table, diag_table, anti_diag_table)


@jax.jit
def kernel(x, absolute_pos_embedding, file_table, rank_table, diag_table, anti_diag_table):
    return _tc_kernel(x, absolute_pos_embedding, file_table, rank_table,
                      diag_table, anti_diag_table)


# speedup vs baseline: 1.4579x; 1.0149x over previous
import jax
import jax.numpy as jnp
from jax.experimental import pallas as pl

D_MODEL = 256
SEQ = 64
BATCH_BLOCK = 128


def _pos_add_body(x_ref, abs_ref, file_ref, rank_ref, diag_ref, anti_ref, o_ref):
    file_emb = jnp.tile(file_ref[...], (8, 1))
    rank_emb = jnp.repeat(rank_ref[...], 8, axis=0)
    row = jax.lax.broadcasted_iota(jnp.int32, (SEQ, 15), 0)
    col = jax.lax.broadcasted_iota(jnp.int32, (SEQ, 15), 1)
    diag_oh = (col == row // 8 + row % 8).astype(jnp.float32)
    anti_oh = (col == row // 8 - row % 8 + 7).astype(jnp.float32)
    diag_emb = jnp.dot(diag_oh, diag_ref[...], preferred_element_type=jnp.float32,
                       precision=jax.lax.Precision.HIGHEST)
    anti_emb = jnp.dot(anti_oh, anti_ref[...], preferred_element_type=jnp.float32,
                       precision=jax.lax.Precision.HIGHEST)
    pos = abs_ref[0] + file_emb + rank_emb + diag_emb + anti_emb
    o_ref[...] = x_ref[...] + pos[None, :, :]


@jax.jit
def kernel(x, absolute_pos_embedding, file_table, rank_table, diag_table, anti_diag_table):
    batch, seq, d = x.shape
    return pl.pallas_call(
        _pos_add_body,
        grid=(batch // BATCH_BLOCK,),
        in_specs=[
            pl.BlockSpec((BATCH_BLOCK, seq, d), lambda i: (i, 0, 0)),
            pl.BlockSpec((1, seq, d), lambda i: (0, 0, 0)),
            pl.BlockSpec((8, d), lambda i: (0, 0)),
            pl.BlockSpec((8, d), lambda i: (0, 0)),
            pl.BlockSpec((15, d), lambda i: (0, 0)),
            pl.BlockSpec((15, d), lambda i: (0, 0)),
        ],
        out_specs=pl.BlockSpec((BATCH_BLOCK, seq, d), lambda i: (i, 0, 0)),
        out_shape=jax.ShapeDtypeStruct(x.shape, x.dtype),
    )(x, absolute_pos_embedding, file_table, rank_table, diag_table, anti_diag_table)
